# baseline (speedup 1.0000x reference)
import functools

import jax
import jax.numpy as jnp
from jax import lax
from jax.experimental import pallas as pl
from jax.experimental.pallas import tpu as pltpu
from jax.experimental.pallas import tpu_sc as plsc


def _sc_workers():
    try:
        info = plsc.get_sparse_core_info()
        return info.num_cores, info.num_subcores
    except Exception:
        return 2, 16


@jax.jit
def _relayout_table(wt):
    nc, ns = _sc_workers()
    nw = nc * ns
    d, v = wt.shape
    n_strips = v // 128
    per_w = (n_strips + nw - 1) // nw
    if per_w % 2 == 0:
        per_w += 1
    extra = n_strips - (per_w - 1) * nw

    mesh = plsc.VectorSubcoreMesh(core_axis_name="c", subcore_axis_name="s")

    @functools.partial(
        pl.kernel,
        out_type=jax.ShapeDtypeStruct((v * d // 128, 128), wt.dtype),
        mesh=mesh,
        scratch_types=[
            pltpu.VMEM((2, d, 128), wt.dtype),
            pltpu.VMEM((2, d, 128), wt.dtype),
            pltpu.SemaphoreType.DMA,
            pltpu.SemaphoreType.DMA,
            pltpu.SemaphoreType.DMA,
            pltpu.SemaphoreType.DMA,
        ],
        compiler_params=pltpu.CompilerParams(
            needs_layout_passes=False, use_tc_tiling_on_sc=True
        ),
    )
    def body(wt_hbm, out_hbm, in_v, tr_v, gi0, gi1, so0, so1):
        wid = lax.axis_index("s") * nc + lax.axis_index("c")
        lo = wid * (per_w - 1) + jnp.minimum(wid, extra)
        gis = (gi0, gi1)
        sos = (so0, so1)
        iota = jax.lax.iota(jnp.int32, 16)

        def strip(j):
            return jnp.minimum(lo + j, n_strips - 1)

        def fire_in(j, b):
            c = strip(j)
            pltpu.async_copy(
                wt_hbm.at[:, pl.ds(c * 128, 128)], in_v.at[b], gis[b]
            )

        def drain_in(b):
            pltpu.make_async_copy(
                wt_hbm.at[:, pl.ds(0, 128)], in_v.at[b], gis[b]
            ).wait()

        def permute(b):
            src = in_v.at[b]
            dst = tr_v.at[b]

            @pl.loop(0, 32)
            def _(p2):
                for k in range(8):
                    row = 16 * (k % 2) + iota
                    col = jnp.full((16,), 4 * p2 + k // 2, dtype=jnp.int32)
                    vals = plsc.load_gather(src, [row, col])
                    dst[p2, pl.ds(16 * k, 16)] = vals

        def fire_out(j, b):
            c = strip(j)
            pltpu.async_copy(tr_v.at[b], out_hbm.at[pl.ds(c * 32, 32)], sos[b])

        def drain_out(b):
            pltpu.make_async_copy(
                tr_v.at[b], out_hbm.at[pl.ds(0, 32)], sos[b]
            ).wait()

        fire_in(0, 0)
        fire_in(1, 1)
        drain_in(0)
        permute(0)
        fire_in(2, 0)
        fire_out(0, 0)

        @pl.loop(0, (per_w - 1) // 2)
        def _(t):
            a = 2 * t + 1
            drain_in(1)

            @pl.when(t > 0)
            def _():
                drain_out(1)

            permute(1)

            @pl.when(a + 2 < per_w)
            def _():
                fire_in(a + 2, 1)

            fire_out(a, 1)

            bb = 2 * t + 2
            drain_in(0)
            drain_out(0)
            permute(0)

            @pl.when(bb + 2 < per_w)
            def _():
                fire_in(bb + 2, 0)

            fire_out(bb, 0)

        drain_out(0)
        drain_out(1)

    return body(wt)


@functools.partial(jax.jit, static_argnames=("h_dim", "b_dim"))
def _gather_t(idx, table, *, h_dim, b_dim):
    nc, ns = _sc_workers()
    nw = nc * ns
    d = table.shape[1]
    n_per_w = idx.shape[1]

    cb = 5
    chunk = cb * 128
    n_chunks = n_per_w // chunk
    blocks_per_w = n_per_w // 128
    n_cb = b_dim // 128
    out_rows = h_dim * (d // 8) * n_cb * 8
    assert n_chunks % 2 == 0 and d % 8 == 0

    mesh = plsc.VectorSubcoreMesh(core_axis_name="c", subcore_axis_name="s")

    @functools.partial(
        pl.kernel,
        out_type=jax.ShapeDtypeStruct((out_rows, 128), table.dtype),
        mesh=mesh,
        scratch_types=[
            pltpu.VMEM((n_per_w,), jnp.int32),
            pltpu.VMEM((2, chunk, d), table.dtype),
            pltpu.VMEM((cb * d, 128), table.dtype),
            pltpu.SemaphoreType.DMA,
            pltpu.SemaphoreType.DMA,
            pltpu.SemaphoreType.DMA,
        ],
        compiler_params=pltpu.CompilerParams(
            needs_layout_passes=False, use_tc_tiling_on_sc=False
        ),
    )
    def body(idx_hbm, table_hbm, out_hbm, idx_v, rows_v, trans_v, g0, g1, st):
        wid = lax.axis_index("s") * nc + lax.axis_index("c")
        base_blk = wid * blocks_per_w

        pltpu.sync_copy(idx_hbm.at[wid], idx_v)

        iota = jax.lax.iota(jnp.int32, 16)

        def fire_gather(g, buf, sem):
            pltpu.async_copy(
                table_hbm.at[idx_v.at[pl.ds(g * chunk, chunk)]],
                rows_v.at[buf],
                sem,
            )

        def drain_gather(buf, sem):
            pltpu.make_async_copy(
                table_hbm.at[idx_v.at[pl.ds(0, chunk)]], rows_v.at[buf], sem
            ).wait()

        def permute_chunk(buf):
            rows = rows_v.at[buf]

            @pl.loop(0, cb * d)
            def _(t):
                blk = t // d
                jc = t - blk * d
                col = jnp.full((16,), jc, dtype=jnp.int32)
                rb = blk * 128
                for k in range(8):
                    row = rb + k * 16 + iota
                    vals = plsc.load_gather(rows, [row, col])
                    trans_v[t, pl.ds(k * 16, 16)] = vals

        def fire_stores(g):
            b0 = base_blk + g * cb
            for blk in range(cb):
                bid = b0 + blk
                h = bid // n_cb
                cbk = bid - h * n_cb
                for r in range(d // 8):
                    rowb = ((h * (d // 8) + r) * n_cb + cbk) * 8
                    pltpu.async_copy(
                        trans_v.at[pl.ds(blk * d + r * 8, 8)],
                        out_hbm.at[pl.ds(rowb, 8)],
                        st,
                    )

        def drain_stores():
            for _ in range(cb * (d // 8)):
                pltpu.make_async_copy(
                    trans_v.at[pl.ds(0, 8)], out_hbm.at[pl.ds(0, 8)], st
                ).wait()

        fire_gather(0, 0, g0)
        fire_gather(1, 1, g1)
        drain_gather(0, g0)
        permute_chunk(0)
        fire_gather(2, 0, g0)
        fire_stores(0)

        @pl.loop(0, n_chunks // 2 - 1)
        def _(t):
            a = 2 * t + 1
            drain_gather(1, g1)
            drain_stores()
            permute_chunk(1)
            fire_gather(a + 2, 1, g1)
            fire_stores(a)

            b = 2 * t + 2
            drain_gather(0, g0)
            drain_stores()
            permute_chunk(0)

            @pl.when(b + 2 < n_chunks)
            def _():
                fire_gather(b + 2, 0, g0)

            fire_stores(b)

        drain_gather(1, g1)
        drain_stores()
        permute_chunk(1)
        fire_stores(n_chunks - 1)
        drain_stores()

    return body(idx, table)


def kernel(indices, W_embedding):
    b, h = indices.shape
    v, d = W_embedding.shape
    nc, ns = _sc_workers()
    nw = nc * ns
    n = b * h
    n_per_w = n // nw
    assert n % nw == 0

    idx_t = indices.T.reshape(nw, n_per_w).astype(jnp.int32)

    w_rm = _relayout_table(W_embedding.T)
    v_full = (v // 128) * 128
    if v_full < v:
        tail = W_embedding[v_full:, :].reshape(-1, 128)
        w_rm = jax.lax.dynamic_update_slice(w_rm, tail, (v_full * d // 128, 0))
    table = w_rm.reshape(v, d)

    out5 = _gather_t(idx_t, table, h_dim=h, b_dim=b)
    out = (
        out5.reshape(h, d // 8, b // 128, 8, 128)
        .transpose(2, 4, 0, 1, 3)
        .reshape(b, h, d)
    )
    return out


# speedup vs baseline: 1.3741x; 1.3741x over previous
import functools

import jax
import jax.numpy as jnp
from jax import lax
from jax.experimental import pallas as pl
from jax.experimental.pallas import tpu as pltpu
from jax.experimental.pallas import tpu_sc as plsc


def _sc_workers():
    try:
        info = plsc.get_sparse_core_info()
        return info.num_cores, info.num_subcores
    except Exception:
        return 2, 16


@jax.jit
def _relayout_table(wt):
    nc, ns = _sc_workers()
    nw = nc * ns
    d, v = wt.shape
    n_strips = v // 128
    per_w = (n_strips + nw - 1) // nw
    if per_w % 2 == 0:
        per_w += 1
    extra = n_strips - (per_w - 1) * nw

    mesh = plsc.VectorSubcoreMesh(core_axis_name="c", subcore_axis_name="s")

    @functools.partial(
        pl.kernel,
        out_type=jax.ShapeDtypeStruct((v * d // 128, 128), wt.dtype),
        mesh=mesh,
        scratch_types=[
            pltpu.VMEM((2, d, 128), wt.dtype),
            pltpu.VMEM((2, d, 128), wt.dtype),
            pltpu.SemaphoreType.DMA,
            pltpu.SemaphoreType.DMA,
            pltpu.SemaphoreType.DMA,
            pltpu.SemaphoreType.DMA,
        ],
        compiler_params=pltpu.CompilerParams(
            needs_layout_passes=False, use_tc_tiling_on_sc=True
        ),
    )
    def body(wt_hbm, out_hbm, in_v, tr_v, gi0, gi1, so0, so1):
        wid = lax.axis_index("s") * nc + lax.axis_index("c")
        lo = wid * (per_w - 1) + jnp.minimum(wid, extra)
        gis = (gi0, gi1)
        sos = (so0, so1)
        iota = jax.lax.iota(jnp.int32, 16)

        def strip(j):
            return jnp.minimum(lo + j, n_strips - 1)

        def fire_in(j, b):
            c = strip(j)
            pltpu.async_copy(
                wt_hbm.at[:, pl.ds(c * 128, 128)], in_v.at[b], gis[b]
            )

        def drain_in(b):
            pltpu.make_async_copy(
                wt_hbm.at[:, pl.ds(0, 128)], in_v.at[b], gis[b]
            ).wait()

        def permute(b):
            src = in_v.at[b]
            dst = tr_v.at[b]

            @pl.loop(0, 32)
            def _(p2):
                base = 4 * p2
                vals = []
                for k in range(8):
                    row = 16 * (k % 2) + iota
                    col = jnp.full((16,), base + k // 2, dtype=jnp.int32)
                    vals.append(plsc.load_gather(src, [row, col]))
                for k in range(8):
                    dst[p2, pl.ds(16 * k, 16)] = vals[k]

        def fire_out(j, b):
            c = strip(j)
            pltpu.async_copy(tr_v.at[b], out_hbm.at[pl.ds(c * 32, 32)], sos[b])

        def drain_out(b):
            pltpu.make_async_copy(
                tr_v.at[b], out_hbm.at[pl.ds(0, 32)], sos[b]
            ).wait()

        fire_in(0, 0)
        fire_in(1, 1)
        drain_in(0)
        permute(0)
        fire_in(2, 0)
        fire_out(0, 0)

        @pl.loop(0, (per_w - 1) // 2)
        def _(t):
            a = 2 * t + 1
            drain_in(1)

            @pl.when(t > 0)
            def _():
                drain_out(1)

            permute(1)

            @pl.when(a + 2 < per_w)
            def _():
                fire_in(a + 2, 1)

            fire_out(a, 1)

            bb = 2 * t + 2
            drain_in(0)
            drain_out(0)
            permute(0)

            @pl.when(bb + 2 < per_w)
            def _():
                fire_in(bb + 2, 0)

            fire_out(bb, 0)

        drain_out(0)
        drain_out(1)

    return body(wt)


@functools.partial(jax.jit, static_argnames=("h_dim", "b_dim"))
def _gather_t(idx, table, *, h_dim, b_dim):
    nc, ns = _sc_workers()
    nw = nc * ns
    d = table.shape[1]
    n_per_w = idx.shape[1]

    cb = 5
    chunk = cb * 128
    n_chunks = n_per_w // chunk
    blocks_per_w = n_per_w // 128
    n_cb = b_dim // 128
    out_rows = h_dim * (d // 8) * n_cb * 8
    assert n_chunks % 2 == 0 and d % 8 == 0

    mesh = plsc.VectorSubcoreMesh(core_axis_name="c", subcore_axis_name="s")

    @functools.partial(
        pl.kernel,
        out_type=jax.ShapeDtypeStruct((out_rows, 128), table.dtype),
        mesh=mesh,
        scratch_types=[
            pltpu.VMEM((n_per_w,), jnp.int32),
            pltpu.VMEM((2, chunk, d), table.dtype),
            pltpu.VMEM((cb * d, 128), table.dtype),
            pltpu.SemaphoreType.DMA,
            pltpu.SemaphoreType.DMA,
            pltpu.SemaphoreType.DMA,
        ],
        compiler_params=pltpu.CompilerParams(
            needs_layout_passes=False, use_tc_tiling_on_sc=False
        ),
    )
    def body(idx_hbm, table_hbm, out_hbm, idx_v, rows_v, trans_v, g0, g1, st):
        wid = lax.axis_index("s") * nc + lax.axis_index("c")
        base_blk = wid * blocks_per_w

        pltpu.sync_copy(idx_hbm.at[wid], idx_v)

        iota = jax.lax.iota(jnp.int32, 16)

        def fire_gather(g, buf, sem):
            pltpu.async_copy(
                table_hbm.at[idx_v.at[pl.ds(g * chunk, chunk)]],
                rows_v.at[buf],
                sem,
            )

        def drain_gather(buf, sem):
            pltpu.make_async_copy(
                table_hbm.at[idx_v.at[pl.ds(0, chunk)]], rows_v.at[buf], sem
            ).wait()

        def permute_chunk(buf):
            rows = rows_v.at[buf]

            @pl.loop(0, cb * d)
            def _(t):
                blk = t // d
                jc = t - blk * d
                col = jnp.full((16,), jc, dtype=jnp.int32)
                rb = blk * 128
                vals = []
                for k in range(8):
                    row = rb + k * 16 + iota
                    vals.append(plsc.load_gather(rows, [row, col]))
                for k in range(8):
                    trans_v[t, pl.ds(k * 16, 16)] = vals[k]

        def fire_stores(g):
            b0 = base_blk + g * cb
            for blk in range(cb):
                bid = b0 + blk
                h = bid // n_cb
                cbk = bid - h * n_cb
                for r in range(d // 8):
                    rowb = ((h * (d // 8) + r) * n_cb + cbk) * 8
                    pltpu.async_copy(
                        trans_v.at[pl.ds(blk * d + r * 8, 8)],
                        out_hbm.at[pl.ds(rowb, 8)],
                        st,
                    )

        def drain_stores():
            for _ in range(cb * (d // 8)):
                pltpu.make_async_copy(
                    trans_v.at[pl.ds(0, 8)], out_hbm.at[pl.ds(0, 8)], st
                ).wait()

        fire_gather(0, 0, g0)
        fire_gather(1, 1, g1)
        drain_gather(0, g0)
        permute_chunk(0)
        fire_gather(2, 0, g0)
        fire_stores(0)

        @pl.loop(0, n_chunks // 2 - 1)
        def _(t):
            a = 2 * t + 1
            drain_gather(1, g1)
            drain_stores()
            permute_chunk(1)
            fire_gather(a + 2, 1, g1)
            fire_stores(a)

            b = 2 * t + 2
            drain_gather(0, g0)
            drain_stores()
            permute_chunk(0)

            @pl.when(b + 2 < n_chunks)
            def _():
                fire_gather(b + 2, 0, g0)

            fire_stores(b)

        drain_gather(1, g1)
        drain_stores()
        permute_chunk(1)
        fire_stores(n_chunks - 1)
        drain_stores()

    return body(idx, table)


def kernel(indices, W_embedding):
    b, h = indices.shape
    v, d = W_embedding.shape
    nc, ns = _sc_workers()
    nw = nc * ns
    n = b * h
    n_per_w = n // nw
    assert n % nw == 0

    idx_t = indices.T.reshape(nw, n_per_w).astype(jnp.int32)

    w_rm = _relayout_table(W_embedding.T)
    v_full = (v // 128) * 128
    if v_full < v:
        tail = W_embedding[v_full:, :].reshape(-1, 128)
        w_rm = jax.lax.dynamic_update_slice(w_rm, tail, (v_full * d // 128, 0))
    table = w_rm.reshape(v, d)

    out5 = _gather_t(idx_t, table, h_dim=h, b_dim=b)
    out = (
        out5.reshape(h, d // 8, b // 128, 8, 128)
        .transpose(2, 4, 0, 1, 3)
        .reshape(b, h, d)
    )
    return out


# speedup vs baseline: 1.3774x; 1.0024x over previous
import functools

import jax
import jax.numpy as jnp
from jax import lax
from jax.experimental import pallas as pl
from jax.experimental.pallas import tpu as pltpu
from jax.experimental.pallas import tpu_sc as plsc


def _sc_workers():
    try:
        info = plsc.get_sparse_core_info()
        return info.num_cores, info.num_subcores
    except Exception:
        return 2, 16


@jax.jit
def _relayout_table(wt):
    nc, ns = _sc_workers()
    nw = nc * ns
    d, v = wt.shape
    n_strips = v // 128
    per_w = (n_strips + nw - 1) // nw
    if per_w % 2 == 0:
        per_w += 1
    extra = n_strips - (per_w - 1) * nw

    mesh = plsc.VectorSubcoreMesh(core_axis_name="c", subcore_axis_name="s")

    @functools.partial(
        pl.kernel,
        out_type=jax.ShapeDtypeStruct((v * d // 128, 128), wt.dtype),
        mesh=mesh,
        scratch_types=[
            pltpu.VMEM((2, d, 128), wt.dtype),
            pltpu.VMEM((2, d, 128), wt.dtype),
            pltpu.SemaphoreType.DMA,
            pltpu.SemaphoreType.DMA,
            pltpu.SemaphoreType.DMA,
            pltpu.SemaphoreType.DMA,
        ],
        compiler_params=pltpu.CompilerParams(
            needs_layout_passes=False, use_tc_tiling_on_sc=True
        ),
    )
    def body(wt_hbm, out_hbm, in_v, tr_v, gi0, gi1, so0, so1):
        wid = lax.axis_index("s") * nc + lax.axis_index("c")
        lo = wid * (per_w - 1) + jnp.minimum(wid, extra)
        gis = (gi0, gi1)
        sos = (so0, so1)
        iota = jax.lax.iota(jnp.int32, 16)

        def strip(j):
            return jnp.minimum(lo + j, n_strips - 1)

        def fire_in(j, b):
            c = strip(j)
            pltpu.async_copy(
                wt_hbm.at[:, pl.ds(c * 128, 128)], in_v.at[b], gis[b]
            )

        def drain_in(b):
            pltpu.make_async_copy(
                wt_hbm.at[:, pl.ds(0, 128)], in_v.at[b], gis[b]
            ).wait()

        def permute(b):
            src = in_v.at[b]
            dst = tr_v.at[b]

            @pl.loop(0, 16)
            def _(u):
                p0 = 2 * u
                base = jnp.full((16,), 4 * p0, dtype=jnp.int32)
                vals = []
                for pp in range(2):
                    for k in range(8):
                        row = 16 * (k % 2) + iota
                        col = base + (4 * pp + k // 2)
                        vals.append(plsc.load_gather(src, [row, col]))
                for pp in range(2):
                    for k in range(8):
                        dst[p0 + pp, pl.ds(16 * k, 16)] = vals[pp * 8 + k]

        def fire_out(j, b):
            c = strip(j)
            pltpu.async_copy(tr_v.at[b], out_hbm.at[pl.ds(c * 32, 32)], sos[b])

        def drain_out(b):
            pltpu.make_async_copy(
                tr_v.at[b], out_hbm.at[pl.ds(0, 32)], sos[b]
            ).wait()

        fire_in(0, 0)
        fire_in(1, 1)
        drain_in(0)
        permute(0)
        fire_in(2, 0)
        fire_out(0, 0)

        @pl.loop(0, (per_w - 1) // 2)
        def _(t):
            a = 2 * t + 1
            drain_in(1)

            @pl.when(t > 0)
            def _():
                drain_out(1)

            permute(1)

            @pl.when(a + 2 < per_w)
            def _():
                fire_in(a + 2, 1)

            fire_out(a, 1)

            bb = 2 * t + 2
            drain_in(0)
            drain_out(0)
            permute(0)

            @pl.when(bb + 2 < per_w)
            def _():
                fire_in(bb + 2, 0)

            fire_out(bb, 0)

        drain_out(0)
        drain_out(1)

    return body(wt)


@functools.partial(jax.jit, static_argnames=("h_dim", "b_dim"))
def _gather_t(idx, table, *, h_dim, b_dim):
    nc, ns = _sc_workers()
    nw = nc * ns
    d = table.shape[1]
    n_per_w = idx.shape[1]

    cb = 5
    chunk = cb * 128
    n_chunks = n_per_w // chunk
    blocks_per_w = n_per_w // 128
    n_cb = b_dim // 128
    out_rows = h_dim * (d // 8) * n_cb * 8
    assert n_chunks % 2 == 0 and d % 8 == 0

    mesh = plsc.VectorSubcoreMesh(core_axis_name="c", subcore_axis_name="s")

    @functools.partial(
        pl.kernel,
        out_type=jax.ShapeDtypeStruct((out_rows, 128), table.dtype),
        mesh=mesh,
        scratch_types=[
            pltpu.VMEM((n_per_w,), jnp.int32),
            pltpu.VMEM((2, chunk, d), table.dtype),
            pltpu.VMEM((cb * d, 128), table.dtype),
            pltpu.SemaphoreType.DMA,
            pltpu.SemaphoreType.DMA,
            pltpu.SemaphoreType.DMA,
        ],
        compiler_params=pltpu.CompilerParams(
            needs_layout_passes=False, use_tc_tiling_on_sc=False
        ),
    )
    def body(idx_hbm, table_hbm, out_hbm, idx_v, rows_v, trans_v, g0, g1, st):
        wid = lax.axis_index("s") * nc + lax.axis_index("c")
        base_blk = wid * blocks_per_w

        pltpu.sync_copy(idx_hbm.at[wid], idx_v)

        iota = jax.lax.iota(jnp.int32, 16)

        def fire_gather(g, buf, sem):
            pltpu.async_copy(
                table_hbm.at[idx_v.at[pl.ds(g * chunk, chunk)]],
                rows_v.at[buf],
                sem,
            )

        def drain_gather(buf, sem):
            pltpu.make_async_copy(
                table_hbm.at[idx_v.at[pl.ds(0, chunk)]], rows_v.at[buf], sem
            ).wait()

        def permute_chunk(buf):
            rows = rows_v.at[buf]
            for blk in range(cb):
                rb = blk * 128
                tb = blk * d

                @pl.loop(0, d // 2)
                def _(u):
                    j0 = 2 * u
                    c0 = jnp.full((16,), j0, dtype=jnp.int32)
                    vals = []
                    for jj in range(2):
                        for k in range(8):
                            row = rb + k * 16 + iota
                            vals.append(
                                plsc.load_gather(rows, [row, c0 + jj])
                            )
                    t0 = tb + j0
                    for jj in range(2):
                        for k in range(8):
                            trans_v[t0 + jj, pl.ds(k * 16, 16)] = vals[
                                jj * 8 + k
                            ]

        def fire_stores(g):
            b0 = base_blk + g * cb
            for blk in range(cb):
                bid = b0 + blk
                h = bid // n_cb
                cbk = bid - h * n_cb
                for r in range(d // 8):
                    rowb = ((h * (d // 8) + r) * n_cb + cbk) * 8
                    pltpu.async_copy(
                        trans_v.at[pl.ds(blk * d + r * 8, 8)],
                        out_hbm.at[pl.ds(rowb, 8)],
                        st,
                    )

        def drain_stores():
            for _ in range(cb * (d // 8)):
                pltpu.make_async_copy(
                    trans_v.at[pl.ds(0, 8)], out_hbm.at[pl.ds(0, 8)], st
                ).wait()

        fire_gather(0, 0, g0)
        fire_gather(1, 1, g1)
        drain_gather(0, g0)
        permute_chunk(0)
        fire_gather(2, 0, g0)
        fire_stores(0)

        @pl.loop(0, n_chunks // 2 - 1)
        def _(t):
            a = 2 * t + 1
            drain_gather(1, g1)
            drain_stores()
            permute_chunk(1)
            fire_gather(a + 2, 1, g1)
            fire_stores(a)

            b = 2 * t + 2
            drain_gather(0, g0)
            drain_stores()
            permute_chunk(0)

            @pl.when(b + 2 < n_chunks)
            def _():
                fire_gather(b + 2, 0, g0)

            fire_stores(b)

        drain_gather(1, g1)
        drain_stores()
        permute_chunk(1)
        fire_stores(n_chunks - 1)
        drain_stores()

    return body(idx, table)


def kernel(indices, W_embedding):
    b, h = indices.shape
    v, d = W_embedding.shape
    nc, ns = _sc_workers()
    nw = nc * ns
    n = b * h
    n_per_w = n // nw
    assert n % nw == 0

    idx_t = indices.T.reshape(nw, n_per_w).astype(jnp.int32)

    w_rm = _relayout_table(W_embedding.T)
    v_full = (v // 128) * 128
    if v_full < v:
        tail = W_embedding[v_full:, :].reshape(-1, 128)
        w_rm = jax.lax.dynamic_update_slice(w_rm, tail, (v_full * d // 128, 0))
    table = w_rm.reshape(v, d)

    out5 = _gather_t(idx_t, table, h_dim=h, b_dim=b)
    out = (
        out5.reshape(h, d // 8, b // 128, 8, 128)
        .transpose(2, 4, 0, 1, 3)
        .reshape(b, h, d)
    )
    return out


# speedup vs baseline: 2.2562x; 1.6380x over previous
import functools

import jax
import jax.numpy as jnp
from jax import lax
from jax.experimental import pallas as pl
from jax.experimental.pallas import tpu as pltpu
from jax.experimental.pallas import tpu_sc as plsc


def _sc_workers():
    try:
        info = plsc.get_sparse_core_info()
        return info.num_cores, info.num_subcores
    except Exception:
        return 2, 16


@jax.jit
def _relayout_table(wt):
    nc, ns = _sc_workers()
    nw = nc * ns
    d, v = wt.shape
    n_strips = v // 128
    per_w = (n_strips + nw - 1) // nw
    if per_w % 2 == 0:
        per_w += 1
    extra = n_strips - (per_w - 1) * nw

    mesh = plsc.VectorSubcoreMesh(core_axis_name="c", subcore_axis_name="s")

    @functools.partial(
        pl.kernel,
        out_type=jax.ShapeDtypeStruct((v * d // 128, 128), wt.dtype),
        mesh=mesh,
        scratch_types=[
            pltpu.VMEM((2, d, 128), wt.dtype),
            pltpu.VMEM((2, d, 128), wt.dtype),
            pltpu.SemaphoreType.DMA,
            pltpu.SemaphoreType.DMA,
            pltpu.SemaphoreType.DMA,
            pltpu.SemaphoreType.DMA,
        ],
        compiler_params=pltpu.CompilerParams(
            needs_layout_passes=False, use_tc_tiling_on_sc=True
        ),
    )
    def body(wt_hbm, out_hbm, in_v, tr_v, gi0, gi1, so0, so1):
        wid = lax.axis_index("s") * nc + lax.axis_index("c")
        lo = wid * (per_w - 1) + jnp.minimum(wid, extra)
        gis = (gi0, gi1)
        sos = (so0, so1)
        iota = jax.lax.iota(jnp.int32, 16)

        def strip(j):
            return jnp.minimum(lo + j, n_strips - 1)

        def fire_in(j, b):
            c = strip(j)
            pltpu.async_copy(
                wt_hbm.at[:, pl.ds(c * 128, 128)], in_v.at[b], gis[b]
            )

        def drain_in(b):
            pltpu.make_async_copy(
                wt_hbm.at[:, pl.ds(0, 128)], in_v.at[b], gis[b]
            ).wait()

        def permute(b):
            src = in_v.at[b]
            dst = tr_v.at[b]

            @pl.loop(0, 16)
            def _(u):
                p0 = 2 * u
                base = jnp.full((16,), 4 * p0, dtype=jnp.int32)
                vals = []
                for pp in range(2):
                    for k in range(8):
                        row = 16 * (k % 2) + iota
                        col = base + (4 * pp + k // 2)
                        vals.append(plsc.load_gather(src, [row, col]))
                for pp in range(2):
                    for k in range(8):
                        dst[p0 + pp, pl.ds(16 * k, 16)] = vals[pp * 8 + k]

        def fire_out(j, b):
            c = strip(j)
            pltpu.async_copy(tr_v.at[b], out_hbm.at[pl.ds(c * 32, 32)], sos[b])

        def drain_out(b):
            pltpu.make_async_copy(
                tr_v.at[b], out_hbm.at[pl.ds(0, 32)], sos[b]
            ).wait()

        fire_in(0, 0)
        fire_in(1, 1)
        drain_in(0)
        permute(0)
        fire_in(2, 0)
        fire_out(0, 0)

        @pl.loop(0, (per_w - 1) // 2)
        def _(t):
            a = 2 * t + 1
            drain_in(1)

            @pl.when(t > 0)
            def _():
                drain_out(1)

            permute(1)

            @pl.when(a + 2 < per_w)
            def _():
                fire_in(a + 2, 1)

            fire_out(a, 1)

            bb = 2 * t + 2
            drain_in(0)
            drain_out(0)
            permute(0)

            @pl.when(bb + 2 < per_w)
            def _():
                fire_in(bb + 2, 0)

            fire_out(bb, 0)

        drain_out(0)
        drain_out(1)

    return body(wt)


@functools.partial(jax.jit, static_argnames=("h_dim", "b_dim"))
def _gather_t(idx, table, *, h_dim, b_dim):
    nc, ns = _sc_workers()
    nw = nc * ns
    d = table.shape[1]
    n_per_w = idx.shape[1]

    cb = 5
    chunk = cb * 128
    n_chunks = n_per_w // chunk
    blocks_per_w = n_per_w // 128
    n_cb = b_dim // 128
    out_rows = h_dim * (d // 8) * n_cb * 8
    assert n_chunks % 2 == 0 and d % 8 == 0

    mesh = plsc.VectorSubcoreMesh(core_axis_name="c", subcore_axis_name="s")

    @functools.partial(
        pl.kernel,
        out_type=jax.ShapeDtypeStruct((out_rows, 128), table.dtype),
        mesh=mesh,
        scratch_types=[
            pltpu.VMEM((n_per_w,), jnp.int32),
            pltpu.VMEM((2, chunk, d), table.dtype),
            pltpu.VMEM((cb * d, 129), table.dtype),
            pltpu.SemaphoreType.DMA,
            pltpu.SemaphoreType.DMA,
            pltpu.SemaphoreType.DMA,
        ],
        compiler_params=pltpu.CompilerParams(
            needs_layout_passes=False, use_tc_tiling_on_sc=False
        ),
    )
    def body(idx_hbm, table_hbm, out_hbm, idx_v, rows_v, trans_v, g0, g1, st):
        wid = lax.axis_index("s") * nc + lax.axis_index("c")
        base_blk = wid * blocks_per_w

        pltpu.sync_copy(idx_hbm.at[wid], idx_v)

        iota = jax.lax.iota(jnp.int32, 16)

        def fire_gather(g, buf, sem):
            pltpu.async_copy(
                table_hbm.at[idx_v.at[pl.ds(g * chunk, chunk)]],
                rows_v.at[buf],
                sem,
            )

        def drain_gather(buf, sem):
            pltpu.make_async_copy(
                table_hbm.at[idx_v.at[pl.ds(0, chunk)]], rows_v.at[buf], sem
            ).wait()

        def permute_chunk(buf):
            rows = rows_v.at[buf]
            for blk in range(cb):
                rb = blk * 128
                tb = blk * d

                @pl.loop(0, 64)
                def _(u):
                    bis = [2 * u, 2 * u + 1]
                    vals = []
                    for bb in range(2):
                        n = rb + bis[bb]
                        for k in range(d // 16):
                            vals.append(rows[n, pl.ds(16 * k, 16)])
                    for bb in range(2):
                        col = jnp.full((16,), bis[bb], dtype=jnp.int32)
                        for k in range(d // 16):
                            rowv = tb + 16 * k + iota
                            plsc.store_scatter(
                                trans_v,
                                [rowv, col],
                                vals[bb * (d // 16) + k],
                            )

        def fire_stores(g):
            b0 = base_blk + g * cb
            for blk in range(cb):
                bid = b0 + blk
                h = bid // n_cb
                cbk = bid - h * n_cb
                for r in range(d // 8):
                    rowb = ((h * (d // 8) + r) * n_cb + cbk) * 8
                    pltpu.async_copy(
                        trans_v.at[pl.ds(blk * d + r * 8, 8), pl.ds(0, 128)],
                        out_hbm.at[pl.ds(rowb, 8)],
                        st,
                    )

        def drain_stores():
            for _ in range(cb * (d // 8)):
                pltpu.make_async_copy(
                    trans_v.at[pl.ds(0, 8), pl.ds(0, 128)],
                    out_hbm.at[pl.ds(0, 8)],
                    st,
                ).wait()

        fire_gather(0, 0, g0)
        fire_gather(1, 1, g1)
        drain_gather(0, g0)
        permute_chunk(0)
        fire_gather(2, 0, g0)
        fire_stores(0)

        @pl.loop(0, n_chunks // 2 - 1)
        def _(t):
            a = 2 * t + 1
            drain_gather(1, g1)
            drain_stores()
            permute_chunk(1)
            fire_gather(a + 2, 1, g1)
            fire_stores(a)

            b = 2 * t + 2
            drain_gather(0, g0)
            drain_stores()
            permute_chunk(0)

            @pl.when(b + 2 < n_chunks)
            def _():
                fire_gather(b + 2, 0, g0)

            fire_stores(b)

        drain_gather(1, g1)
        drain_stores()
        permute_chunk(1)
        fire_stores(n_chunks - 1)
        drain_stores()

    return body(idx, table)


def kernel(indices, W_embedding):
    b, h = indices.shape
    v, d = W_embedding.shape
    nc, ns = _sc_workers()
    nw = nc * ns
    n = b * h
    n_per_w = n // nw
    assert n % nw == 0

    idx_t = indices.T.reshape(nw, n_per_w).astype(jnp.int32)

    w_rm = _relayout_table(W_embedding.T)
    v_full = (v // 128) * 128
    if v_full < v:
        tail = W_embedding[v_full:, :].reshape(-1, 128)
        w_rm = jax.lax.dynamic_update_slice(w_rm, tail, (v_full * d // 128, 0))
    table = w_rm.reshape(v, d)

    out5 = _gather_t(idx_t, table, h_dim=h, b_dim=b)
    out = (
        out5.reshape(h, d // 8, b // 128, 8, 128)
        .transpose(2, 4, 0, 1, 3)
        .reshape(b, h, d)
    )
    return out


# speedup vs baseline: 4.8008x; 2.1279x over previous
import functools

import jax
import jax.numpy as jnp
from jax import lax
from jax.experimental import pallas as pl
from jax.experimental.pallas import tpu as pltpu
from jax.experimental.pallas import tpu_sc as plsc


def _sc_workers():
    try:
        info = plsc.get_sparse_core_info()
        return info.num_cores, info.num_subcores
    except Exception:
        return 2, 16


@jax.jit
def _relayout_table(wt):
    nc, ns = _sc_workers()
    nw = nc * ns
    d, v = wt.shape
    n_strips = v // 128
    per_w = (n_strips + nw - 1) // nw
    if per_w % 2 == 0:
        per_w += 1
    extra = n_strips - (per_w - 1) * nw

    mesh = plsc.VectorSubcoreMesh(core_axis_name="c", subcore_axis_name="s")

    @functools.partial(
        pl.kernel,
        out_type=jax.ShapeDtypeStruct((v * d // 128, 128), wt.dtype),
        mesh=mesh,
        scratch_types=[
            pltpu.VMEM((2, d, 128), wt.dtype),
            pltpu.VMEM((2, d, 128), wt.dtype),
            pltpu.SemaphoreType.DMA,
            pltpu.SemaphoreType.DMA,
            pltpu.SemaphoreType.DMA,
            pltpu.SemaphoreType.DMA,
        ],
        compiler_params=pltpu.CompilerParams(
            needs_layout_passes=False, use_tc_tiling_on_sc=True
        ),
    )
    def body(wt_hbm, out_hbm, in_v, tr_v, gi0, gi1, so0, so1):
        wid = lax.axis_index("s") * nc + lax.axis_index("c")
        lo = wid * (per_w - 1) + jnp.minimum(wid, extra)
        gis = (gi0, gi1)
        sos = (so0, so1)
        iota = jax.lax.iota(jnp.int32, 16)

        def strip(j):
            return jnp.minimum(lo + j, n_strips - 1)

        def fire_in(j, b):
            c = strip(j)
            pltpu.async_copy(
                wt_hbm.at[:, pl.ds(c * 128, 128)], in_v.at[b], gis[b]
            )

        def drain_in(b):
            pltpu.make_async_copy(
                wt_hbm.at[:, pl.ds(0, 128)], in_v.at[b], gis[b]
            ).wait()

        def permute(b):
            src = in_v.at[b]
            dst = tr_v.at[b]

            @pl.loop(0, 16)
            def _(s):
                d15 = (iota + s) & 15
                vals = []
                for k in range(8):
                    colv = 16 * k + d15
                    for jh in range(d // 16):
                        rowv = 16 * jh + iota
                        vals.append(plsc.load_gather(src, [rowv, colv]))
                for k in range(8):
                    colv = 16 * k + d15
                    p2v = colv >> 2
                    cv2 = ((colv & 3) << 5) + iota
                    for jh in range(d // 16):
                        plsc.store_scatter(
                            dst,
                            [p2v, cv2 + 16 * jh],
                            vals[k * (d // 16) + jh],
                        )

        def fire_out(j, b):
            c = strip(j)
            pltpu.async_copy(tr_v.at[b], out_hbm.at[pl.ds(c * 32, 32)], sos[b])

        def drain_out(b):
            pltpu.make_async_copy(
                tr_v.at[b], out_hbm.at[pl.ds(0, 32)], sos[b]
            ).wait()

        fire_in(0, 0)
        fire_in(1, 1)
        drain_in(0)
        permute(0)
        fire_in(2, 0)
        fire_out(0, 0)

        @pl.loop(0, (per_w - 1) // 2)
        def _(t):
            a = 2 * t + 1
            drain_in(1)

            @pl.when(t > 0)
            def _():
                drain_out(1)

            permute(1)

            @pl.when(a + 2 < per_w)
            def _():
                fire_in(a + 2, 1)

            fire_out(a, 1)

            bb = 2 * t + 2
            drain_in(0)
            drain_out(0)
            permute(0)

            @pl.when(bb + 2 < per_w)
            def _():
                fire_in(bb + 2, 0)

            fire_out(bb, 0)

        drain_out(0)
        drain_out(1)

    return body(wt)


@functools.partial(jax.jit, static_argnames=("h_dim", "b_dim"))
def _gather_t(idx, table, *, h_dim, b_dim):
    nc, ns = _sc_workers()
    nw = nc * ns
    d = table.shape[1]
    n_per_w = idx.shape[1]

    cb = 5
    chunk = cb * 128
    n_chunks = n_per_w // chunk
    blocks_per_w = n_per_w // 128
    n_cb = b_dim // 128
    out_rows = h_dim * (d // 8) * n_cb * 8
    assert n_chunks % 2 == 0 and d % 8 == 0

    mesh = plsc.VectorSubcoreMesh(core_axis_name="c", subcore_axis_name="s")

    @functools.partial(
        pl.kernel,
        out_type=jax.ShapeDtypeStruct((out_rows, 128), table.dtype),
        mesh=mesh,
        scratch_types=[
            pltpu.VMEM((n_per_w,), jnp.int32),
            pltpu.VMEM((2, chunk, d), table.dtype),
            pltpu.VMEM((cb * d, 129), table.dtype),
            pltpu.SemaphoreType.DMA,
            pltpu.SemaphoreType.DMA,
            pltpu.SemaphoreType.DMA,
        ],
        compiler_params=pltpu.CompilerParams(
            needs_layout_passes=False, use_tc_tiling_on_sc=False
        ),
    )
    def body(idx_hbm, table_hbm, out_hbm, idx_v, rows_v, trans_v, g0, g1, st):
        wid = lax.axis_index("s") * nc + lax.axis_index("c")
        base_blk = wid * blocks_per_w

        pltpu.sync_copy(idx_hbm.at[wid], idx_v)

        iota = jax.lax.iota(jnp.int32, 16)

        def fire_gather(g, buf, sem):
            pltpu.async_copy(
                table_hbm.at[idx_v.at[pl.ds(g * chunk, chunk)]],
                rows_v.at[buf],
                sem,
            )

        def drain_gather(buf, sem):
            pltpu.make_async_copy(
                table_hbm.at[idx_v.at[pl.ds(0, chunk)]], rows_v.at[buf], sem
            ).wait()

        def permute_chunk(buf):
            rows = rows_v.at[buf]
            for blk in range(cb):
                rb = blk * 128
                tb = blk * d

                @pl.loop(0, 64)
                def _(u):
                    bis = [2 * u, 2 * u + 1]
                    vals = []
                    for bb in range(2):
                        n = rb + bis[bb]
                        for k in range(d // 16):
                            vals.append(rows[n, pl.ds(16 * k, 16)])
                    for bb in range(2):
                        col = jnp.full((16,), bis[bb], dtype=jnp.int32)
                        for k in range(d // 16):
                            rowv = tb + 16 * k + iota
                            plsc.store_scatter(
                                trans_v,
                                [rowv, col],
                                vals[bb * (d // 16) + k],
                            )

        def fire_stores(g):
            b0 = base_blk + g * cb
            for blk in range(cb):
                bid = b0 + blk
                h = bid // n_cb
                cbk = bid - h * n_cb
                for r in range(d // 8):
                    rowb = ((h * (d // 8) + r) * n_cb + cbk) * 8
                    pltpu.async_copy(
                        trans_v.at[pl.ds(blk * d + r * 8, 8), pl.ds(0, 128)],
                        out_hbm.at[pl.ds(rowb, 8)],
                        st,
                    )

        def drain_stores():
            for _ in range(cb * (d // 8)):
                pltpu.make_async_copy(
                    trans_v.at[pl.ds(0, 8), pl.ds(0, 128)],
                    out_hbm.at[pl.ds(0, 8)],
                    st,
                ).wait()

        fire_gather(0, 0, g0)
        fire_gather(1, 1, g1)
        drain_gather(0, g0)
        permute_chunk(0)
        fire_gather(2, 0, g0)
        fire_stores(0)

        @pl.loop(0, n_chunks // 2 - 1)
        def _(t):
            a = 2 * t + 1
            drain_gather(1, g1)
            drain_stores()
            permute_chunk(1)
            fire_gather(a + 2, 1, g1)
            fire_stores(a)

            b = 2 * t + 2
            drain_gather(0, g0)
            drain_stores()
            permute_chunk(0)

            @pl.when(b + 2 < n_chunks)
            def _():
                fire_gather(b + 2, 0, g0)

            fire_stores(b)

        drain_gather(1, g1)
        drain_stores()
        permute_chunk(1)
        fire_stores(n_chunks - 1)
        drain_stores()

    return body(idx, table)


def kernel(indices, W_embedding):
    b, h = indices.shape
    v, d = W_embedding.shape
    nc, ns = _sc_workers()
    nw = nc * ns
    n = b * h
    n_per_w = n // nw
    assert n % nw == 0

    idx_t = indices.T.reshape(nw, n_per_w).astype(jnp.int32)

    w_rm = _relayout_table(W_embedding.T)
    v_full = (v // 128) * 128
    if v_full < v:
        tail = W_embedding[v_full:, :].reshape(-1, 128)
        w_rm = jax.lax.dynamic_update_slice(w_rm, tail, (v_full * d // 128, 0))
    table = w_rm.reshape(v, d)

    out5 = _gather_t(idx_t, table, h_dim=h, b_dim=b)
    out = (
        out5.reshape(h, d // 8, b // 128, 8, 128)
        .transpose(2, 4, 0, 1, 3)
        .reshape(b, h, d)
    )
    return out
